# Initial kernel scaffold; baseline (speedup 1.0000x reference)
import jax
import jax.numpy as jnp
from jax.experimental import pallas as pl


def kernel(x, edge_index, deg, W_backbone, b_backbone, Binary_param):
    raise NotImplementedError("write your pallas kernel here")


# speedup vs baseline: 13.5476x; 13.5476x over previous
import functools

import jax
import jax.numpy as jnp
from jax import lax
from jax.experimental import pallas as pl
from jax.experimental.pallas import tpu as pltpu
from jax.experimental.pallas import tpu_sc as plsc

_N, _D, _C, _E = 10000, 128, 16, 320000

_NC, _NS = 2, 16
_NW = _NC * _NS
_CH = 80
_EPW = 10240
_NCH = _EPW // _CH
_E_PAD = _NW * _EPW
_N_PAD = 10112
_RPS = _N_PAD // _NS
_NB = 8


def _prep_body(x_ref, w_ref, b_ref, deg_ref, bp_ref,
               unary_ref, xred_ref, m_ref, binary_ref, b2_ref):
    x = x_ref[...]
    w = w_ref[...]
    b = b_ref[...]
    deg = deg_ref[...]
    bp = bp_ref[...]
    unary = jnp.dot(x, w, preferred_element_type=jnp.float32) + b
    xred = unary / deg
    binary = (bp + bp.T) * 0.5
    b2 = binary * 0.5
    unary_ref[...] = unary
    xred_ref[...] = xred
    binary_ref[...] = binary
    b2_ref[...] = b2
    mx = jnp.max(xred, axis=1, keepdims=True)
    bmax = jnp.max(b2)
    e = jnp.exp(xred - mx)
    eb = jnp.exp(b2 - bmax)
    s = jnp.dot(e, eb, preferred_element_type=jnp.float32)
    m_ref[...] = jnp.log(s) + mx + bmax


def _prep_call(x, w, b2d, deg, bp):
    bn = 2000
    grid = _N // bn
    return pl.pallas_call(
        _prep_body,
        grid=(grid,),
        in_specs=[
            pl.BlockSpec((bn, _D), lambda i: (i, 0)),
            pl.BlockSpec((_D, _C), lambda i: (0, 0)),
            pl.BlockSpec((1, _C), lambda i: (0, 0)),
            pl.BlockSpec((bn, 1), lambda i: (i, 0)),
            pl.BlockSpec((_C, _C), lambda i: (0, 0)),
        ],
        out_specs=[
            pl.BlockSpec((bn, _C), lambda i: (i, 0)),
            pl.BlockSpec((bn, _C), lambda i: (i, 0)),
            pl.BlockSpec((bn, _C), lambda i: (i, 0)),
            pl.BlockSpec((_C, _C), lambda i: (0, 0)),
            pl.BlockSpec((_C, _C), lambda i: (0, 0)),
        ],
        out_shape=[
            jax.ShapeDtypeStruct((_N, _C), jnp.float32),
            jax.ShapeDtypeStruct((_N, _C), jnp.float32),
            jax.ShapeDtypeStruct((_N, _C), jnp.float32),
            jax.ShapeDtypeStruct((_C, _C), jnp.float32),
            jax.ShapeDtypeStruct((_C, _C), jnp.float32),
        ],
    )(x, w, b2d, deg, bp)


def _sc_body(m_hbm, src_hbm, dst_hbm, zero_hbm, out_hbm,
             src_v, dst_v, rows_v, acc_sh, gsem):
    cid = lax.axis_index("c")
    sid = lax.axis_index("s")
    wid = cid * _NS + sid
    pltpu.sync_copy(zero_hbm.at[pl.ds(sid * _RPS, _RPS)],
                    acc_sh.at[pl.ds(sid * _RPS, _RPS)])
    pltpu.sync_copy(src_hbm.at[wid], src_v)
    pltpu.sync_copy(dst_hbm.at[wid], dst_v)
    plsc.subcore_barrier()

    for b in range(_NB):
        pltpu.async_copy(m_hbm.at[dst_v.at[b]], rows_v.at[b], gsem)

    def outer(g, _):
        j0 = g * _NB
        for b in range(_NB):
            j = j0 + b
            pltpu.make_async_copy(m_hbm.at[dst_v.at[j]], rows_v.at[b],
                                  gsem).wait()
            pltpu.sync_copy(rows_v.at[b], acc_sh.at[src_v.at[j]], add=True)

            @pl.when(j + _NB < _NCH)
            def _():
                pltpu.async_copy(m_hbm.at[dst_v.at[j + _NB]], rows_v.at[b],
                                 gsem)
        return 0

    lax.fori_loop(0, _NCH // _NB, outer, 0)
    plsc.subcore_barrier()
    pltpu.sync_copy(acc_sh.at[pl.ds(sid * _RPS, _RPS)],
                    out_hbm.at[cid, pl.ds(sid * _RPS, _RPS)])


_sc_call = functools.partial(
    pl.kernel,
    out_type=jax.ShapeDtypeStruct((_NC, _N_PAD, _C), jnp.float32),
    mesh=plsc.VectorSubcoreMesh(core_axis_name="c", subcore_axis_name="s"),
    compiler_params=pltpu.CompilerParams(use_tc_tiling_on_sc=False),
    scratch_types=[
        pltpu.VMEM((_NCH, _CH), jnp.int32),
        pltpu.VMEM((_NCH, _CH), jnp.int32),
        pltpu.VMEM((_NB, _CH, _C), jnp.float32),
        pltpu.VMEM_SHARED((_N_PAD, _C), jnp.float32),
        pltpu.SemaphoreType.DMA,
    ],
)(_sc_body)


def _final_body(xred_ref, aggr_ref, out_ref):
    a = aggr_ref[...]
    t = xred_ref[...] + a[0] + a[1]
    mx = jnp.max(t, axis=1, keepdims=True)
    out_ref[...] = mx + jnp.log(
        jnp.sum(jnp.exp(t - mx), axis=1, keepdims=True))


def _final_call(xred, aggr2):
    bn = 2000
    grid = _N // bn
    return pl.pallas_call(
        _final_body,
        grid=(grid,),
        in_specs=[
            pl.BlockSpec((bn, _C), lambda i: (i, 0)),
            pl.BlockSpec((_NC, bn, _C), lambda i: (0, i, 0)),
        ],
        out_specs=pl.BlockSpec((bn, 1), lambda i: (i, 0)),
        out_shape=jax.ShapeDtypeStruct((_N, 1), jnp.float32),
    )(xred, aggr2)


def kernel(x, edge_index, deg, W_backbone, b_backbone, Binary_param):
    unary, xred, m_tab, binary, b2 = _prep_call(
        x, W_backbone, b_backbone.reshape(1, _C), deg, Binary_param)

    src = edge_index[0]
    dst = edge_index[1]
    pad = _E_PAD - _E
    src_p = jnp.concatenate(
        [src, jnp.full((pad,), _N, jnp.int32)]).reshape(_NW, _NCH, _CH)
    dst_p = jnp.concatenate(
        [dst, jnp.zeros((pad,), jnp.int32)]).reshape(_NW, _NCH, _CH)
    zeros = jnp.zeros((_N_PAD, _C), jnp.float32)

    aggr2 = _sc_call(m_tab, src_p, dst_p, zeros)

    log_z = _final_call(xred, aggr2[:, :_N, :]).reshape(_N)
    return (unary, xred, binary, b2, log_z)


# speedup vs baseline: 14.0016x; 1.0335x over previous
import functools

import jax
import jax.numpy as jnp
from jax import lax
from jax.experimental import pallas as pl
from jax.experimental.pallas import tpu as pltpu
from jax.experimental.pallas import tpu_sc as plsc

_N, _D, _C, _E = 10000, 128, 16, 320000

_NC, _NS = 2, 16
_NW = _NC * _NS
_CH = 128
_EPW = 10240
_NCH = _EPW // _CH
_E_PAD = _NW * _EPW
_N_PAD = 10112
_RPS = _N_PAD // _NS
_NB = 4
_R = 2 * _NB


def _prep_body(x_ref, w_ref, b_ref, deg_ref, bp_ref,
               unary_ref, xred_ref, m_ref, binary_ref, b2_ref):
    x = x_ref[...]
    w = w_ref[...]
    b = b_ref[...]
    deg = deg_ref[...]
    bp = bp_ref[...]
    unary = jnp.dot(x, w, preferred_element_type=jnp.float32) + b
    xred = unary / deg
    binary = (bp + bp.T) * 0.5
    b2 = binary * 0.5
    unary_ref[...] = unary
    xred_ref[...] = xred
    binary_ref[...] = binary
    b2_ref[...] = b2
    mx = jnp.max(xred, axis=1, keepdims=True)
    bmax = jnp.max(b2)
    e = jnp.exp(xred - mx)
    eb = jnp.exp(b2 - bmax)
    s = jnp.dot(e, eb, preferred_element_type=jnp.float32)
    m_ref[...] = jnp.log(s) + mx + bmax


def _prep_call(x, w, b2d, deg, bp):
    bn = 2000
    grid = _N // bn
    return pl.pallas_call(
        _prep_body,
        grid=(grid,),
        in_specs=[
            pl.BlockSpec((bn, _D), lambda i: (i, 0)),
            pl.BlockSpec((_D, _C), lambda i: (0, 0)),
            pl.BlockSpec((1, _C), lambda i: (0, 0)),
            pl.BlockSpec((bn, 1), lambda i: (i, 0)),
            pl.BlockSpec((_C, _C), lambda i: (0, 0)),
        ],
        out_specs=[
            pl.BlockSpec((bn, _C), lambda i: (i, 0)),
            pl.BlockSpec((bn, _C), lambda i: (i, 0)),
            pl.BlockSpec((bn, _C), lambda i: (i, 0)),
            pl.BlockSpec((_C, _C), lambda i: (0, 0)),
            pl.BlockSpec((_C, _C), lambda i: (0, 0)),
        ],
        out_shape=[
            jax.ShapeDtypeStruct((_N, _C), jnp.float32),
            jax.ShapeDtypeStruct((_N, _C), jnp.float32),
            jax.ShapeDtypeStruct((_N, _C), jnp.float32),
            jax.ShapeDtypeStruct((_C, _C), jnp.float32),
            jax.ShapeDtypeStruct((_C, _C), jnp.float32),
        ],
    )(x, w, b2d, deg, bp)


def _sc_body(m_hbm, src_hbm, dst_hbm, zero_hbm, out_hbm,
             src_v, dst_v, rows_v, acc_sh, gsem, ssem):
    cid = lax.axis_index("c")
    sid = lax.axis_index("s")
    wid = cid * _NS + sid
    pltpu.sync_copy(zero_hbm.at[pl.ds(sid * _RPS, _RPS)],
                    acc_sh.at[pl.ds(sid * _RPS, _RPS)])
    pltpu.sync_copy(src_hbm.at[wid], src_v)
    pltpu.sync_copy(dst_hbm.at[wid], dst_v)
    plsc.subcore_barrier()

    for b in range(_NB):
        pltpu.async_copy(m_hbm.at[dst_v.at[b]], rows_v.at[b], gsem.at[b])

    def outer(g, _):
        j0 = g * _R
        for bb in range(_R):
            j = j0 + bb
            pltpu.make_async_copy(m_hbm.at[dst_v.at[j]], rows_v.at[bb],
                                  gsem.at[bb]).wait()
            pltpu.async_copy(rows_v.at[bb], acc_sh.at[src_v.at[j]],
                             ssem.at[bb], add=True)
            b2 = (bb + _NB) % _R

            @pl.when(j + _NB < _NCH)
            def _():
                @pl.when(j >= _NB)
                def _():
                    pltpu.make_async_copy(
                        rows_v.at[b2], acc_sh.at[src_v.at[j - _NB]],
                        ssem.at[b2]).wait()
                pltpu.async_copy(m_hbm.at[dst_v.at[j + _NB]], rows_v.at[b2],
                                 gsem.at[b2])
        return 0

    lax.fori_loop(0, _NCH // _R, outer, 0)
    for bb in range(_R):
        j = _NCH - _R + bb
        pltpu.make_async_copy(rows_v.at[bb], acc_sh.at[src_v.at[j]],
                              ssem.at[bb]).wait()
    plsc.subcore_barrier()
    pltpu.sync_copy(acc_sh.at[pl.ds(sid * _RPS, _RPS)],
                    out_hbm.at[cid, pl.ds(sid * _RPS, _RPS)])


_sc_call = functools.partial(
    pl.kernel,
    out_type=jax.ShapeDtypeStruct((_NC, _N_PAD, _C), jnp.float32),
    mesh=plsc.VectorSubcoreMesh(core_axis_name="c", subcore_axis_name="s"),
    compiler_params=pltpu.CompilerParams(use_tc_tiling_on_sc=False),
    scratch_types=[
        pltpu.VMEM((_NCH, _CH), jnp.int32),
        pltpu.VMEM((_NCH, _CH), jnp.int32),
        pltpu.VMEM((_R, _CH, _C), jnp.float32),
        pltpu.VMEM_SHARED((_N_PAD, _C), jnp.float32),
        pltpu.SemaphoreType.DMA((_R,)),
        pltpu.SemaphoreType.DMA((_R,)),
    ],
)(_sc_body)


def _final_body(xred_ref, aggr_ref, out_ref):
    a = aggr_ref[...]
    t = xred_ref[...] + a[0] + a[1]
    mx = jnp.max(t, axis=1, keepdims=True)
    out_ref[...] = mx + jnp.log(
        jnp.sum(jnp.exp(t - mx), axis=1, keepdims=True))


def _final_call(xred, aggr2):
    bn = 2000
    grid = _N // bn
    return pl.pallas_call(
        _final_body,
        grid=(grid,),
        in_specs=[
            pl.BlockSpec((bn, _C), lambda i: (i, 0)),
            pl.BlockSpec((_NC, bn, _C), lambda i: (0, i, 0)),
        ],
        out_specs=pl.BlockSpec((bn, 1), lambda i: (i, 0)),
        out_shape=jax.ShapeDtypeStruct((_N, 1), jnp.float32),
    )(xred, aggr2)


def kernel(x, edge_index, deg, W_backbone, b_backbone, Binary_param):
    unary, xred, m_tab, binary, b2 = _prep_call(
        x, W_backbone, b_backbone.reshape(1, _C), deg, Binary_param)

    src = edge_index[0]
    dst = edge_index[1]
    pad = _E_PAD - _E
    src_p = jnp.concatenate(
        [src, jnp.full((pad,), _N, jnp.int32)]).reshape(_NW, _NCH, _CH)
    dst_p = jnp.concatenate(
        [dst, jnp.zeros((pad,), jnp.int32)]).reshape(_NW, _NCH, _CH)
    zeros = jnp.zeros((_N_PAD, _C), jnp.float32)

    aggr2 = _sc_call(m_tab, src_p, dst_p, zeros)

    log_z = _final_call(xred, aggr2).reshape(_N)
    return (unary, xred, binary, b2, log_z)
